# baseline (speedup 1.0000x reference)
import functools

import jax
import jax.numpy as jnp
from jax import lax
from jax.experimental import pallas as pl
from jax.experimental.pallas import tpu as pltpu
from jax.experimental.pallas import tpu_sc as plsc

E = 160000
T = 320000
H = 128
IE = 64

NCORE = 2
NSUB = 16
LANES = 16
NPASS = 3
RNG = 26752
ACC = RNG + 128
EP = NPASS * NCORE * RNG
CH = 80
TPT = T // NSUB
GSZ = 400
GCH = GSZ // CH
NGRP = TPT // GSZ
ZROWS = ACC // NSUB
OROWS = RNG // NSUB

BA = 3200
BB = 3200


def _silu(v):
    return v * jax.lax.logistic(v)


def _pre_body(x_ref, rbf_ref, wji_ref, bji_ref, wkj_ref, bkj_ref,
              wr1_ref, wr2_ref, wd_ref, xji_ref, tmp2_ref):
    xb = x_ref[...]
    xji_ref[...] = _silu(jnp.dot(xb, wji_ref[...],
                                 preferred_element_type=jnp.float32) + bji_ref[...])
    t2 = _silu(jnp.dot(xb, wkj_ref[...],
                       preferred_element_type=jnp.float32) + bkj_ref[...])
    r = jnp.dot(jnp.dot(rbf_ref[...], wr1_ref[...],
                        preferred_element_type=jnp.float32), wr2_ref[...],
                preferred_element_type=jnp.float32)
    tmp2_ref[...] = _silu(jnp.dot(t2 * r, wd_ref[...],
                                  preferred_element_type=jnp.float32))


def _sbf_body(sbf_ref, w1_ref, w2_ref, out_ref):
    out_ref[...] = jnp.dot(jnp.dot(sbf_ref[...], w1_ref[...],
                                   preferred_element_type=jnp.float32), w2_ref[...],
                           preferred_element_type=jnp.float32)


def _post_body(xkt_ref, xji_ref, x_ref, wup_ref, wb1_ref, bb1_ref, wb2_ref,
               bb2_ref, wlin_ref, blin_ref, wa1_ref, ba1_ref, wa2_ref,
               ba2_ref, out_ref):
    dot = functools.partial(jnp.dot, preferred_element_type=jnp.float32)
    h0 = xji_ref[...] + _silu(dot(xkt_ref[...], wup_ref[...]))
    h1 = h0 + _silu(dot(_silu(dot(h0, wb1_ref[...]) + bb1_ref[...]),
                        wb2_ref[...]) + bb2_ref[...])
    h2 = _silu(dot(h1, wlin_ref[...]) + blin_ref[...]) + x_ref[...]
    out_ref[...] = h2 + _silu(dot(_silu(dot(h2, wa1_ref[...]) + ba1_ref[...]),
                                  wa2_ref[...]) + ba2_ref[...])


def _wspec(shape):
    return pl.BlockSpec(shape, lambda i: tuple(0 for _ in shape))


def _pre(x, rbf, w_ji, b_ji, w_kj, b_kj, w_r1, w_r2, w_d):
    nr = rbf.shape[1]
    return pl.pallas_call(
        _pre_body,
        grid=(E // BA,),
        in_specs=[
            pl.BlockSpec((BA, H), lambda i: (i, 0)),
            pl.BlockSpec((BA, nr), lambda i: (i, 0)),
            _wspec((H, H)), _wspec((1, H)), _wspec((H, H)), _wspec((1, H)),
            _wspec((nr, 8)), _wspec((8, H)), _wspec((H, IE)),
        ],
        out_specs=[
            pl.BlockSpec((BA, H), lambda i: (i, 0)),
            pl.BlockSpec((BA, IE), lambda i: (i, 0)),
        ],
        out_shape=[
            jax.ShapeDtypeStruct((E, H), jnp.float32),
            jax.ShapeDtypeStruct((E, IE), jnp.float32),
        ],
    )(x, rbf, w_ji, b_ji.reshape(1, H), w_kj, b_kj.reshape(1, H),
      w_r1, w_r2, w_d)


def _sbf_t(sbf, w1, w2):
    k = sbf.shape[1]
    return pl.pallas_call(
        _sbf_body,
        grid=(T // BB,),
        in_specs=[
            pl.BlockSpec((BB, k), lambda i: (i, 0)),
            _wspec((k, 8)), _wspec((8, IE)),
        ],
        out_specs=pl.BlockSpec((BB, IE), lambda i: (i, 0)),
        out_shape=jax.ShapeDtypeStruct((T, IE), jnp.float32),
    )(sbf, w1, w2)


def _post(xkt, xji, x, w_up, w_b1, b_b1, w_b2, b_b2, w_lin, b_lin,
          w_a1, b_a1, w_a2, b_a2):
    return pl.pallas_call(
        _post_body,
        grid=(E // BA,),
        in_specs=[
            pl.BlockSpec((BA, IE), lambda i: (i, 0)),
            pl.BlockSpec((BA, H), lambda i: (i, 0)),
            pl.BlockSpec((BA, H), lambda i: (i, 0)),
            _wspec((IE, H)),
            _wspec((H, H)), _wspec((1, H)), _wspec((H, H)), _wspec((1, H)),
            _wspec((H, H)), _wspec((1, H)),
            _wspec((H, H)), _wspec((1, H)), _wspec((H, H)), _wspec((1, H)),
        ],
        out_specs=pl.BlockSpec((BA, H), lambda i: (i, 0)),
        out_shape=jax.ShapeDtypeStruct((E, H), jnp.float32),
    )(xkt, xji, x, w_up,
      w_b1, b_b1.reshape(1, H), w_b2, b_b2.reshape(1, H),
      w_lin, b_lin.reshape(1, H),
      w_a1, b_a1.reshape(1, H), w_a2, b_a2.reshape(1, H))


def _sc_body(tmp2_hbm, sbft_hbm, kj_hbm, ji_hbm, zero_hbm, out_hbm,
             acc, semg0, semg1, sems0, sems1, semi0, semi1,
             kjg, jig, jilb, rows, sbfb):
    sid = lax.axis_index("s")
    cid = lax.axis_index("c")
    tbase = sid * TPT

    semg = (semg0, semg1)
    sems = (sems0, sems1)
    semi = (semi0, semi1)

    def _idx_pair(g, gb):
        k = pltpu.make_async_copy(
            kj_hbm.at[pl.ds(tbase + g * GSZ, GSZ)], kjg.at[gb], semi[gb])
        j = pltpu.make_async_copy(
            ji_hbm.at[pl.ds(tbase + g * GSZ, GSZ)], jig.at[gb], semi[gb])
        return k, j

    def _gather_pair(g, gb, ci, b):
        gth = pltpu.make_async_copy(
            tmp2_hbm.at[kjg.at[gb, pl.ds(ci * CH, CH)]], rows.at[b], semg[b])
        s = pltpu.make_async_copy(
            sbft_hbm.at[pl.ds(tbase + g * GSZ + ci * CH, CH)],
            sbfb.at[b], sems[b])
        return gth, s

    def _chunk(g, gb, ci, b, base):
        gth, s = _gather_pair(g, gb, ci, b)
        gth.wait()
        s.wait()
        for k in range(CH // LANES):
            ji = jig[gb, pl.ds(ci * CH + k * LANES, LANES)]
            loc = ji - base
            inb = (loc >= 0) & (loc < RNG)
            jilb[b, pl.ds(k * LANES, LANES)] = jnp.where(inb, loc, RNG)

        def _mul(jr, c3):
            for k2 in range(IE // LANES):
                sl = pl.ds(k2 * LANES, LANES)
                rows[b, jr, sl] = rows[b, jr, sl] * sbfb[b, jr, sl]
            return c3

        lax.fori_loop(0, CH, _mul, 0)
        pltpu.sync_copy(rows.at[b], acc.at[jilb.at[b]], add=True)

        ci2, g2 = ci + 2, g
        gb2 = gb
        if ci2 >= GCH:
            ci2, g2, gb2 = ci2 - GCH, g + 1, 1 - gb

        @pl.when(g2 * GCH + ci2 < NCHT)
        def _():
            gth2, s2 = _gather_pair(g2, gb2, ci2, b)
            gth2.start()
            s2.start()

    def _pass(p, carry):
        base = (p * NCORE + cid) * RNG

        pltpu.sync_copy(zero_hbm, acc.at[pl.ds(sid * ZROWS, ZROWS)])
        for d in _idx_pair(0, 0):
            d.start()
        for d in _idx_pair(0, 0):
            d.wait()
        for d in _idx_pair(1, 1):
            d.start()
        for b in (0, 1):
            gth, s = _gather_pair(0, 0, b, b)
            gth.start()
            s.start()
        plsc.subcore_barrier()

        def _group_pair(g2, cg):
            for gb in (0, 1):
                g = 2 * g2 + gb
                @pl.when(g + 1 < NGRP)
                def _():
                    for d in _idx_pair(g + 1, 1 - gb):
                        d.wait()

                for ci in range(GCH):
                    _chunk(g, gb, ci, (GCH * gb + ci) % 2, base)

                @pl.when(g + 2 < NGRP)
                def _():
                    for d in _idx_pair(g + 2, gb):
                        d.start()
            return cg

        lax.fori_loop(0, NGRP // 2, _group_pair, 0)
        plsc.subcore_barrier()
        pltpu.sync_copy(
            acc.at[pl.ds(sid * OROWS, OROWS)],
            out_hbm.at[pl.ds(base + sid * OROWS, OROWS)])
        return carry

    lax.fori_loop(0, NPASS, _pass, 0)


def _sc_body_sync(tmp2_hbm, sbft_hbm, kj_hbm, ji_hbm, zero_hbm, out_hbm,
                  acc, semg0, semg1, sems0, sems1, semi0, semi1,
                  kjg, jig, jilb, rows, sbfb):
    sid = lax.axis_index("s")
    cid = lax.axis_index("c")
    tbase = sid * TPT

    def _pass(p, carry):
        base = (p * NCORE + cid) * RNG
        pltpu.sync_copy(zero_hbm, acc.at[pl.ds(sid * ZROWS, ZROWS)])
        plsc.subcore_barrier()

        def _group(g, cg):
            pltpu.sync_copy(kj_hbm.at[pl.ds(tbase + g * GSZ, GSZ)],
                            kjg.at[0])
            pltpu.sync_copy(ji_hbm.at[pl.ds(tbase + g * GSZ, GSZ)],
                            jig.at[0])

            def _chunk(ci, c2):
                pltpu.sync_copy(
                    tmp2_hbm.at[kjg.at[0, pl.ds(ci * CH, CH)]], rows.at[0])
                pltpu.sync_copy(
                    sbft_hbm.at[pl.ds(tbase + g * GSZ + ci * CH, CH)],
                    sbfb.at[0])
                for k in range(CH // LANES):
                    ji = jig[0, pl.ds(ci * CH + k * LANES, LANES)]
                    loc = ji - base
                    inb = (loc >= 0) & (loc < RNG)
                    jilb[0, pl.ds(k * LANES, LANES)] = jnp.where(inb, loc, RNG)

                def _mul(jr, c3):
                    for k2 in range(IE // LANES):
                        sl = pl.ds(k2 * LANES, LANES)
                        rows[0, jr, sl] = rows[0, jr, sl] * sbfb[0, jr, sl]
                    return c3

                lax.fori_loop(0, CH, _mul, 0)
                pltpu.sync_copy(rows.at[0], acc.at[jilb.at[0]], add=True)
                return c2

            lax.fori_loop(0, GCH, _chunk, 0)
            return cg

        lax.fori_loop(0, NGRP, _group, 0)
        plsc.subcore_barrier()
        pltpu.sync_copy(
            acc.at[pl.ds(sid * OROWS, OROWS)],
            out_hbm.at[pl.ds(base + sid * OROWS, OROWS)])
        return carry

    lax.fori_loop(0, NPASS, _pass, 0)


@functools.partial(
    pl.kernel,
    mesh=plsc.VectorSubcoreMesh(core_axis_name="c", subcore_axis_name="s"),
    compiler_params=pltpu.CompilerParams(use_tc_tiling_on_sc=False),
    out_type=jax.ShapeDtypeStruct((EP, IE), jnp.float32),
    scratch_types=[
        pltpu.VMEM_SHARED((ACC, IE), jnp.float32),
        pltpu.SemaphoreType.DMA,
        pltpu.SemaphoreType.DMA,
        pltpu.SemaphoreType.DMA,
        pltpu.SemaphoreType.DMA,
        pltpu.SemaphoreType.DMA,
        pltpu.SemaphoreType.DMA,
    ],
)
def _sc_gather_mul_scatter(tmp2_hbm, sbft_hbm, kj_hbm, ji_hbm, zero_hbm,
                           out_hbm, *rest):
    pl.run_scoped(
        lambda kjg, jig, jilb, rows, sbfb: _sc_body_sync(
            tmp2_hbm, sbft_hbm, kj_hbm, ji_hbm, zero_hbm, out_hbm,
            *rest, kjg, jig, jilb, rows, sbfb),
        pltpu.VMEM((2, GSZ), jnp.int32),
        pltpu.VMEM((2, GSZ), jnp.int32),
        pltpu.VMEM((2, CH), jnp.int32),
        pltpu.VMEM((2, CH, IE), jnp.float32),
        pltpu.VMEM((2, CH, IE), jnp.float32),
    )


def _gather_mul_scatter(tmp2, sbf_t, idx_kj, idx_ji):
    zero = jnp.zeros((ZROWS, IE), jnp.float32)
    out = _sc_gather_mul_scatter(tmp2, sbf_t, idx_kj, idx_ji, zero)
    return out[:E]


def kernel(x, rbf, sbf, idx_kj, idx_ji, bt, lambda_d, alpha,
           W_rbf1, W_rbf2, W_sbf1, W_sbf2, W_kj, b_kj, W_ji, b_ji,
           W_down, W_up, W_b1, b_b1, W_b2, b_b2, W_lin, b_lin,
           W_a1, b_a1, W_a2, b_a2):
    a32 = jnp.asarray(alpha, jnp.float32)
    xji, tmp2 = _pre(x, rbf, W_ji, b_ji, W_kj[-1], b_kj[-1],
                     W_rbf1[-1], W_rbf2[-1], W_down[-1])
    sbf_t = _sbf_t(sbf, W_sbf1[-1], W_sbf2[-1] * a32)
    xkt = _gather_mul_scatter(tmp2, sbf_t, idx_kj, idx_ji)
    return _post(xkt, xji, x, W_up, W_b1, b_b1, W_b2, b_b2,
                 W_lin, b_lin, W_a1, b_a1, W_a2, b_a2)


# speedup vs baseline: 1.0000x; 1.0000x over previous
import functools

import jax
import jax.numpy as jnp
from jax import lax
from jax.experimental import pallas as pl
from jax.experimental.pallas import tpu as pltpu
from jax.experimental.pallas import tpu_sc as plsc

E = 160000
T = 320000
H = 128
IE = 64

NCORE = 2
NSUB = 16
LANES = 16
NPASS = 3
RNG = 26752
ACC = RNG + 128
EP = NPASS * NCORE * RNG
CH = 80
TPT = T // NSUB
GSZ = 400
GCH = GSZ // CH
NGRP = TPT // GSZ
ZROWS = ACC // NSUB
OROWS = RNG // NSUB

BA = 3200
BB = 3200


def _silu(v):
    return v * jax.lax.logistic(v)


def _pre_body(x_ref, rbf_ref, wji_ref, bji_ref, wkj_ref, bkj_ref,
              wr1_ref, wr2_ref, wd_ref, xji_ref, tmp2_ref):
    xb = x_ref[...]
    xji_ref[...] = _silu(jnp.dot(xb, wji_ref[...],
                                 preferred_element_type=jnp.float32) + bji_ref[...])
    t2 = _silu(jnp.dot(xb, wkj_ref[...],
                       preferred_element_type=jnp.float32) + bkj_ref[...])
    r = jnp.dot(jnp.dot(rbf_ref[...], wr1_ref[...],
                        preferred_element_type=jnp.float32), wr2_ref[...],
                preferred_element_type=jnp.float32)
    tmp2_ref[...] = _silu(jnp.dot(t2 * r, wd_ref[...],
                                  preferred_element_type=jnp.float32))


def _sbf_body(sbf_ref, w1_ref, w2_ref, out_ref):
    out_ref[...] = jnp.dot(jnp.dot(sbf_ref[...], w1_ref[...],
                                   preferred_element_type=jnp.float32), w2_ref[...],
                           preferred_element_type=jnp.float32)


def _post_body(xkt_ref, xji_ref, x_ref, wup_ref, wb1_ref, bb1_ref, wb2_ref,
               bb2_ref, wlin_ref, blin_ref, wa1_ref, ba1_ref, wa2_ref,
               ba2_ref, out_ref):
    dot = functools.partial(jnp.dot, preferred_element_type=jnp.float32)
    h0 = xji_ref[...] + _silu(dot(xkt_ref[...], wup_ref[...]))
    h1 = h0 + _silu(dot(_silu(dot(h0, wb1_ref[...]) + bb1_ref[...]),
                        wb2_ref[...]) + bb2_ref[...])
    h2 = _silu(dot(h1, wlin_ref[...]) + blin_ref[...]) + x_ref[...]
    out_ref[...] = h2 + _silu(dot(_silu(dot(h2, wa1_ref[...]) + ba1_ref[...]),
                                  wa2_ref[...]) + ba2_ref[...])


def _wspec(shape):
    return pl.BlockSpec(shape, lambda i: tuple(0 for _ in shape))


def _pre(x, rbf, w_ji, b_ji, w_kj, b_kj, w_r1, w_r2, w_d):
    nr = rbf.shape[1]
    return pl.pallas_call(
        _pre_body,
        grid=(E // BA,),
        in_specs=[
            pl.BlockSpec((BA, H), lambda i: (i, 0)),
            pl.BlockSpec((BA, nr), lambda i: (i, 0)),
            _wspec((H, H)), _wspec((1, H)), _wspec((H, H)), _wspec((1, H)),
            _wspec((nr, 8)), _wspec((8, H)), _wspec((H, IE)),
        ],
        out_specs=[
            pl.BlockSpec((BA, H), lambda i: (i, 0)),
            pl.BlockSpec((BA, IE), lambda i: (i, 0)),
        ],
        out_shape=[
            jax.ShapeDtypeStruct((E, H), jnp.float32),
            jax.ShapeDtypeStruct((E, IE), jnp.float32),
        ],
    )(x, rbf, w_ji, b_ji.reshape(1, H), w_kj, b_kj.reshape(1, H),
      w_r1, w_r2, w_d)


def _sbf_t(sbf, w1, w2):
    k = sbf.shape[1]
    return pl.pallas_call(
        _sbf_body,
        grid=(T // BB,),
        in_specs=[
            pl.BlockSpec((BB, k), lambda i: (i, 0)),
            _wspec((k, 8)), _wspec((8, IE)),
        ],
        out_specs=pl.BlockSpec((BB, IE), lambda i: (i, 0)),
        out_shape=jax.ShapeDtypeStruct((T, IE), jnp.float32),
    )(sbf, w1, w2)


def _post(xkt, xji, x, w_up, w_b1, b_b1, w_b2, b_b2, w_lin, b_lin,
          w_a1, b_a1, w_a2, b_a2):
    return pl.pallas_call(
        _post_body,
        grid=(E // BA,),
        in_specs=[
            pl.BlockSpec((BA, IE), lambda i: (i, 0)),
            pl.BlockSpec((BA, H), lambda i: (i, 0)),
            pl.BlockSpec((BA, H), lambda i: (i, 0)),
            _wspec((IE, H)),
            _wspec((H, H)), _wspec((1, H)), _wspec((H, H)), _wspec((1, H)),
            _wspec((H, H)), _wspec((1, H)),
            _wspec((H, H)), _wspec((1, H)), _wspec((H, H)), _wspec((1, H)),
        ],
        out_specs=pl.BlockSpec((BA, H), lambda i: (i, 0)),
        out_shape=jax.ShapeDtypeStruct((E, H), jnp.float32),
    )(xkt, xji, x, w_up,
      w_b1, b_b1.reshape(1, H), w_b2, b_b2.reshape(1, H),
      w_lin, b_lin.reshape(1, H),
      w_a1, b_a1.reshape(1, H), w_a2, b_a2.reshape(1, H))


def _sc_body(tmp2_hbm, sbft_hbm, kj_hbm, ji_hbm, zero_hbm, out_hbm,
             acc, semg0, semg1, sems0, sems1, semi0, semi1,
             kjg, jig, jilb, rows, sbfb):
    sid = lax.axis_index("s")
    cid = lax.axis_index("c")
    tbase = sid * TPT

    semg = (semg0, semg1)
    sems = (sems0, sems1)
    semi = (semi0, semi1)

    def _idx_pair(g, gb):
        k = pltpu.make_async_copy(
            kj_hbm.at[pl.ds(tbase + g * GSZ, GSZ)], kjg.at[gb], semi[gb])
        j = pltpu.make_async_copy(
            ji_hbm.at[pl.ds(tbase + g * GSZ, GSZ)], jig.at[gb], semi[gb])
        return k, j

    def _gather_pair(g, gb, ci, b):
        gth = pltpu.make_async_copy(
            tmp2_hbm.at[kjg.at[gb, pl.ds(ci * CH, CH)]], rows.at[b], semg[b])
        s = pltpu.make_async_copy(
            sbft_hbm.at[pl.ds(tbase + g * GSZ + ci * CH, CH)],
            sbfb.at[b], sems[b])
        return gth, s

    def _chunk(g, gb, ci, b, base):
        gth, s = _gather_pair(g, gb, ci, b)
        gth.wait()
        s.wait()
        for k in range(CH // LANES):
            ji = jig[gb, pl.ds(ci * CH + k * LANES, LANES)]
            loc = ji - base
            inb = (loc >= 0) & (loc < RNG)
            jilb[b, pl.ds(k * LANES, LANES)] = jnp.where(inb, loc, RNG)

        @plsc.parallel_loop(0, CH, 2, unroll=2)
        def _mul(jr):
            for jo in range(2):
                for k2 in range(IE // LANES):
                    sl = pl.ds(k2 * LANES, LANES)
                    rows[b, jr + jo, sl] = (
                        rows[b, jr + jo, sl] * sbfb[b, jr + jo, sl])

        pltpu.sync_copy(rows.at[b], acc.at[jilb.at[b]], add=True)

        ci2, g2 = ci + 2, g
        gb2 = gb
        if ci2 >= GCH:
            ci2, g2, gb2 = ci2 - GCH, g + 1, 1 - gb

        @pl.when(g2 * GCH + ci2 < NCHT)
        def _():
            gth2, s2 = _gather_pair(g2, gb2, ci2, b)
            gth2.start()
            s2.start()

    def _pass(p, carry):
        base = (p * NCORE + cid) * RNG

        pltpu.sync_copy(zero_hbm, acc.at[pl.ds(sid * ZROWS, ZROWS)])
        for d in _idx_pair(0, 0):
            d.start()
        for d in _idx_pair(0, 0):
            d.wait()
        for d in _idx_pair(1, 1):
            d.start()
        for b in (0, 1):
            gth, s = _gather_pair(0, 0, b, b)
            gth.start()
            s.start()
        plsc.subcore_barrier()

        def _group_pair(g2, cg):
            for gb in (0, 1):
                g = 2 * g2 + gb
                @pl.when(g + 1 < NGRP)
                def _():
                    for d in _idx_pair(g + 1, 1 - gb):
                        d.wait()

                for ci in range(GCH):
                    _chunk(g, gb, ci, (GCH * gb + ci) % 2, base)

                @pl.when(g + 2 < NGRP)
                def _():
                    for d in _idx_pair(g + 2, gb):
                        d.start()
            return cg

        lax.fori_loop(0, NGRP // 2, _group_pair, 0)
        plsc.subcore_barrier()
        pltpu.sync_copy(
            acc.at[pl.ds(sid * OROWS, OROWS)],
            out_hbm.at[pl.ds(base + sid * OROWS, OROWS)])
        return carry

    lax.fori_loop(0, NPASS, _pass, 0)


def _sc_body_sync(tmp2_hbm, sbft_hbm, kj_hbm, ji_hbm, zero_hbm, out_hbm,
                  acc, semg0, semg1, sems0, sems1, semi0, semi1,
                  kjg, jig, jilb, rows, sbfb):
    sid = lax.axis_index("s")
    cid = lax.axis_index("c")
    tbase = sid * TPT

    def _pass(p, carry):
        base = (p * NCORE + cid) * RNG
        pltpu.sync_copy(zero_hbm, acc.at[pl.ds(sid * ZROWS, ZROWS)])
        plsc.subcore_barrier()

        def _group(g, cg):
            pltpu.sync_copy(kj_hbm.at[pl.ds(tbase + g * GSZ, GSZ)],
                            kjg.at[0])
            pltpu.sync_copy(ji_hbm.at[pl.ds(tbase + g * GSZ, GSZ)],
                            jig.at[0])

            def _chunk(ci, c2):
                pltpu.sync_copy(
                    tmp2_hbm.at[kjg.at[0, pl.ds(ci * CH, CH)]], rows.at[0])
                pltpu.sync_copy(
                    sbft_hbm.at[pl.ds(tbase + g * GSZ + ci * CH, CH)],
                    sbfb.at[0])
                for k in range(CH // LANES):
                    ji = jig[0, pl.ds(ci * CH + k * LANES, LANES)]
                    loc = ji - base
                    inb = (loc >= 0) & (loc < RNG)
                    jilb[0, pl.ds(k * LANES, LANES)] = jnp.where(inb, loc, RNG)

                def _mul(jr, c3):
                    for k2 in range(IE // LANES):
                        sl = pl.ds(k2 * LANES, LANES)
                        rows[0, jr, sl] = rows[0, jr, sl] * sbfb[0, jr, sl]
                    return c3

                lax.fori_loop(0, CH, _mul, 0)
                pltpu.sync_copy(rows.at[0], acc.at[jilb.at[0]], add=True)
                return c2

            lax.fori_loop(0, GCH, _chunk, 0)
            return cg

        lax.fori_loop(0, NGRP, _group, 0)
        plsc.subcore_barrier()
        pltpu.sync_copy(
            acc.at[pl.ds(sid * OROWS, OROWS)],
            out_hbm.at[pl.ds(base + sid * OROWS, OROWS)])
        return carry

    lax.fori_loop(0, NPASS, _pass, 0)


@functools.partial(
    pl.kernel,
    mesh=plsc.VectorSubcoreMesh(core_axis_name="c", subcore_axis_name="s"),
    compiler_params=pltpu.CompilerParams(use_tc_tiling_on_sc=False),
    out_type=jax.ShapeDtypeStruct((EP, IE), jnp.float32),
    scratch_types=[
        pltpu.VMEM_SHARED((ACC, IE), jnp.float32),
        pltpu.SemaphoreType.DMA,
        pltpu.SemaphoreType.DMA,
        pltpu.SemaphoreType.DMA,
        pltpu.SemaphoreType.DMA,
        pltpu.SemaphoreType.DMA,
        pltpu.SemaphoreType.DMA,
    ],
)
def _sc_gather_mul_scatter(tmp2_hbm, sbft_hbm, kj_hbm, ji_hbm, zero_hbm,
                           out_hbm, *rest):
    pl.run_scoped(
        lambda kjg, jig, jilb, rows, sbfb: _sc_body_sync(
            tmp2_hbm, sbft_hbm, kj_hbm, ji_hbm, zero_hbm, out_hbm,
            *rest, kjg, jig, jilb, rows, sbfb),
        pltpu.VMEM((2, GSZ), jnp.int32),
        pltpu.VMEM((2, GSZ), jnp.int32),
        pltpu.VMEM((2, CH), jnp.int32),
        pltpu.VMEM((2, CH, IE), jnp.float32),
        pltpu.VMEM((2, CH, IE), jnp.float32),
    )


def _gather_mul_scatter(tmp2, sbf_t, idx_kj, idx_ji):
    zero = jnp.zeros((ZROWS, IE), jnp.float32)
    out = _sc_gather_mul_scatter(tmp2, sbf_t, idx_kj, idx_ji, zero)
    return out[:E]


def kernel(x, rbf, sbf, idx_kj, idx_ji, bt, lambda_d, alpha,
           W_rbf1, W_rbf2, W_sbf1, W_sbf2, W_kj, b_kj, W_ji, b_ji,
           W_down, W_up, W_b1, b_b1, W_b2, b_b2, W_lin, b_lin,
           W_a1, b_a1, W_a2, b_a2):
    a32 = jnp.asarray(alpha, jnp.float32)
    xji, tmp2 = _pre(x, rbf, W_ji, b_ji, W_kj[-1], b_kj[-1],
                     W_rbf1[-1], W_rbf2[-1], W_down[-1])
    sbf_t = _sbf_t(sbf, W_sbf1[-1], W_sbf2[-1] * a32)
    xkt = _gather_mul_scatter(tmp2, sbf_t, idx_kj, idx_ji)
    return _post(xkt, xji, x, W_up, W_b1, b_b1, W_b2, b_b2,
                 W_lin, b_lin, W_a1, b_a1, W_a2, b_a2)


# speedup vs baseline: 1.4083x; 1.4082x over previous
import functools

import jax
import jax.numpy as jnp
from jax import lax
from jax.experimental import pallas as pl
from jax.experimental.pallas import tpu as pltpu
from jax.experimental.pallas import tpu_sc as plsc

E = 160000
T = 320000
H = 128
IE = 64

NCORE = 2
NSUB = 16
LANES = 16
NPASS = 3
RNG = 26752
ACC = RNG + 128
EP = NPASS * NCORE * RNG
CH = 80
TPT = T // NSUB
GSZ = 400
GCH = GSZ // CH
NGRP = TPT // GSZ
NCHT = TPT // CH
ZROWS = ACC // NSUB
OROWS = RNG // NSUB

BA = 3200
BB = 3200


def _silu(v):
    return v * jax.lax.logistic(v)


def _pre_body(x_ref, rbf_ref, wji_ref, bji_ref, wkj_ref, bkj_ref,
              wr1_ref, wr2_ref, wd_ref, xji_ref, tmp2_ref):
    xb = x_ref[...]
    xji_ref[...] = _silu(jnp.dot(xb, wji_ref[...],
                                 preferred_element_type=jnp.float32) + bji_ref[...])
    t2 = _silu(jnp.dot(xb, wkj_ref[...],
                       preferred_element_type=jnp.float32) + bkj_ref[...])
    r = jnp.dot(jnp.dot(rbf_ref[...], wr1_ref[...],
                        preferred_element_type=jnp.float32), wr2_ref[...],
                preferred_element_type=jnp.float32)
    tmp2_ref[...] = _silu(jnp.dot(t2 * r, wd_ref[...],
                                  preferred_element_type=jnp.float32))


def _sbf_body(sbf_ref, w1_ref, w2_ref, out_ref):
    out_ref[...] = jnp.dot(jnp.dot(sbf_ref[...], w1_ref[...],
                                   preferred_element_type=jnp.float32), w2_ref[...],
                           preferred_element_type=jnp.float32)


def _post_body(xkt_ref, xji_ref, x_ref, wup_ref, wb1_ref, bb1_ref, wb2_ref,
               bb2_ref, wlin_ref, blin_ref, wa1_ref, ba1_ref, wa2_ref,
               ba2_ref, out_ref):
    dot = functools.partial(jnp.dot, preferred_element_type=jnp.float32)
    h0 = xji_ref[...] + _silu(dot(xkt_ref[...], wup_ref[...]))
    h1 = h0 + _silu(dot(_silu(dot(h0, wb1_ref[...]) + bb1_ref[...]),
                        wb2_ref[...]) + bb2_ref[...])
    h2 = _silu(dot(h1, wlin_ref[...]) + blin_ref[...]) + x_ref[...]
    out_ref[...] = h2 + _silu(dot(_silu(dot(h2, wa1_ref[...]) + ba1_ref[...]),
                                  wa2_ref[...]) + ba2_ref[...])


def _wspec(shape):
    return pl.BlockSpec(shape, lambda i: tuple(0 for _ in shape))


def _pre(x, rbf, w_ji, b_ji, w_kj, b_kj, w_r1, w_r2, w_d):
    nr = rbf.shape[1]
    return pl.pallas_call(
        _pre_body,
        grid=(E // BA,),
        in_specs=[
            pl.BlockSpec((BA, H), lambda i: (i, 0)),
            pl.BlockSpec((BA, nr), lambda i: (i, 0)),
            _wspec((H, H)), _wspec((1, H)), _wspec((H, H)), _wspec((1, H)),
            _wspec((nr, 8)), _wspec((8, H)), _wspec((H, IE)),
        ],
        out_specs=[
            pl.BlockSpec((BA, H), lambda i: (i, 0)),
            pl.BlockSpec((BA, IE), lambda i: (i, 0)),
        ],
        out_shape=[
            jax.ShapeDtypeStruct((E, H), jnp.float32),
            jax.ShapeDtypeStruct((E, IE), jnp.float32),
        ],
    )(x, rbf, w_ji, b_ji.reshape(1, H), w_kj, b_kj.reshape(1, H),
      w_r1, w_r2, w_d)


def _sbf_t(sbf, w1, w2):
    k = sbf.shape[1]
    return pl.pallas_call(
        _sbf_body,
        grid=(T // BB,),
        in_specs=[
            pl.BlockSpec((BB, k), lambda i: (i, 0)),
            _wspec((k, 8)), _wspec((8, IE)),
        ],
        out_specs=pl.BlockSpec((BB, IE), lambda i: (i, 0)),
        out_shape=jax.ShapeDtypeStruct((T, IE), jnp.float32),
    )(sbf, w1, w2)


def _post(xkt, xji, x, w_up, w_b1, b_b1, w_b2, b_b2, w_lin, b_lin,
          w_a1, b_a1, w_a2, b_a2):
    return pl.pallas_call(
        _post_body,
        grid=(E // BA,),
        in_specs=[
            pl.BlockSpec((BA, IE), lambda i: (i, 0)),
            pl.BlockSpec((BA, H), lambda i: (i, 0)),
            pl.BlockSpec((BA, H), lambda i: (i, 0)),
            _wspec((IE, H)),
            _wspec((H, H)), _wspec((1, H)), _wspec((H, H)), _wspec((1, H)),
            _wspec((H, H)), _wspec((1, H)),
            _wspec((H, H)), _wspec((1, H)), _wspec((H, H)), _wspec((1, H)),
        ],
        out_specs=pl.BlockSpec((BA, H), lambda i: (i, 0)),
        out_shape=jax.ShapeDtypeStruct((E, H), jnp.float32),
    )(xkt, xji, x, w_up,
      w_b1, b_b1.reshape(1, H), w_b2, b_b2.reshape(1, H),
      w_lin, b_lin.reshape(1, H),
      w_a1, b_a1.reshape(1, H), w_a2, b_a2.reshape(1, H))


def _sc_body(tmp2_hbm, sbft_hbm, kj_hbm, ji_hbm, zero_hbm, out_hbm,
             acc, semg0, semg1, sems0, sems1, semi0, semi1, semsc0, semsc1,
             kjg, jig, jilb, rows, sbfb):
    sid = lax.axis_index("s")
    cid = lax.axis_index("c")
    tbase = sid * TPT

    semg = (semg0, semg1)
    sems = (sems0, sems1)
    semi = (semi0, semi1)
    semsc = (semsc0, semsc1)

    def _idx_pair(g, gb):
        k = pltpu.make_async_copy(
            kj_hbm.at[pl.ds(tbase + g * GSZ, GSZ)], kjg.at[gb], semi[gb])
        j = pltpu.make_async_copy(
            ji_hbm.at[pl.ds(tbase + g * GSZ, GSZ)], jig.at[gb], semi[gb])
        return k, j

    def _gather(g, gb, ci, b):
        return pltpu.make_async_copy(
            tmp2_hbm.at[kjg.at[gb, pl.ds(ci * CH, CH)]], rows.at[b], semg[b])

    def _sbf(g, ci, b):
        return pltpu.make_async_copy(
            sbft_hbm.at[pl.ds(tbase + g * GSZ + ci * CH, CH)],
            sbfb.at[b], sems[b])

    def _scatter(b):
        return pltpu.make_async_copy(sbfb.at[b], acc.at[jilb.at[b]],
                                     semsc[b])

    def _nxt(g, gb, ci, d):
        ci2, g2, gb2 = ci + d, g, gb
        while ci2 >= GCH:
            ci2, g2, gb2 = ci2 - GCH, g2 + 1, 1 - gb2
        return g2, gb2, ci2

    def _chunk(g, gb, ci, b, base):
        t = g * GCH + ci
        gth = _gather(g, gb, ci, b)
        gth.wait()
        for k in range(CH // LANES):
            ji = jig[gb, pl.ds(ci * CH + k * LANES, LANES)]
            loc = ji - base
            inb = (loc >= 0) & (loc < RNG)
            jilb[b, pl.ds(k * LANES, LANES)] = jnp.where(inb, loc, RNG)
        _sbf(g, ci, b).wait()

        @plsc.parallel_loop(0, CH, 2, unroll=2)
        def _mul(jr):
            for jo in range(2):
                for k2 in range(IE // LANES):
                    sl = pl.ds(k2 * LANES, LANES)
                    sbfb[b, jr + jo, sl] = (
                        rows[b, jr + jo, sl] * sbfb[b, jr + jo, sl])

        _scatter(b).start(add=True)
        g2, gb2, ci2 = _nxt(g, gb, ci, 2)

        @pl.when(t + 2 < NCHT)
        def _():
            _gather(g2, gb2, ci2, b).start()

        @pl.when(t >= 1)
        def _():
            _scatter(1 - b).wait()
        g1, _gb1, ci1 = _nxt(g, gb, ci, 1)

        @pl.when(t + 1 < NCHT)
        def _():
            _sbf(g1, ci1, 1 - b).start()

    def _pass(p, carry):
        base = (p * NCORE + cid) * RNG

        pltpu.sync_copy(zero_hbm, acc.at[pl.ds(sid * ZROWS, ZROWS)])
        for d in _idx_pair(0, 0):
            d.start()
        for d in _idx_pair(0, 0):
            d.wait()
        for d in _idx_pair(1, 1):
            d.start()
        _gather(0, 0, 0, 0).start()
        _gather(0, 0, 1, 1).start()
        _sbf(0, 0, 0).start()
        plsc.subcore_barrier()

        def _group_pair(g2, cg):
            for gb in (0, 1):
                g = 2 * g2 + gb
                @pl.when(g + 1 < NGRP)
                def _():
                    for d in _idx_pair(g + 1, 1 - gb):
                        d.wait()

                for ci in range(GCH):
                    _chunk(g, gb, ci, (GCH * gb + ci) % 2, base)

                @pl.when(g + 2 < NGRP)
                def _():
                    for d in _idx_pair(g + 2, gb):
                        d.start()
            return cg

        lax.fori_loop(0, NGRP // 2, _group_pair, 0)
        _scatter(1).wait()
        plsc.subcore_barrier()
        pltpu.sync_copy(
            acc.at[pl.ds(sid * OROWS, OROWS)],
            out_hbm.at[pl.ds(base + sid * OROWS, OROWS)])
        return carry

    lax.fori_loop(0, NPASS, _pass, 0)


@functools.partial(
    pl.kernel,
    mesh=plsc.VectorSubcoreMesh(core_axis_name="c", subcore_axis_name="s"),
    compiler_params=pltpu.CompilerParams(use_tc_tiling_on_sc=False),
    out_type=jax.ShapeDtypeStruct((EP, IE), jnp.float32),
    scratch_types=[
        pltpu.VMEM_SHARED((ACC, IE), jnp.float32),
        pltpu.SemaphoreType.DMA,
        pltpu.SemaphoreType.DMA,
        pltpu.SemaphoreType.DMA,
        pltpu.SemaphoreType.DMA,
        pltpu.SemaphoreType.DMA,
        pltpu.SemaphoreType.DMA,
        pltpu.SemaphoreType.DMA,
        pltpu.SemaphoreType.DMA,
    ],
)
def _sc_gather_mul_scatter(tmp2_hbm, sbft_hbm, kj_hbm, ji_hbm, zero_hbm,
                           out_hbm, *rest):
    pl.run_scoped(
        lambda kjg, jig, jilb, rows, sbfb: _sc_body(
            tmp2_hbm, sbft_hbm, kj_hbm, ji_hbm, zero_hbm, out_hbm,
            *rest, kjg, jig, jilb, rows, sbfb),
        pltpu.VMEM((2, GSZ), jnp.int32),
        pltpu.VMEM((2, GSZ), jnp.int32),
        pltpu.VMEM((2, CH), jnp.int32),
        pltpu.VMEM((2, CH, IE), jnp.float32),
        pltpu.VMEM((2, CH, IE), jnp.float32),
    )


def _gather_mul_scatter(tmp2, sbf_t, idx_kj, idx_ji):
    zero = jnp.zeros((ZROWS, IE), jnp.float32)
    out = _sc_gather_mul_scatter(tmp2, sbf_t, idx_kj, idx_ji, zero)
    return out[:E]


def kernel(x, rbf, sbf, idx_kj, idx_ji, bt, lambda_d, alpha,
           W_rbf1, W_rbf2, W_sbf1, W_sbf2, W_kj, b_kj, W_ji, b_ji,
           W_down, W_up, W_b1, b_b1, W_b2, b_b2, W_lin, b_lin,
           W_a1, b_a1, W_a2, b_a2):
    a32 = jnp.asarray(alpha, jnp.float32)
    xji, tmp2 = _pre(x, rbf, W_ji, b_ji, W_kj[-1], b_kj[-1],
                     W_rbf1[-1], W_rbf2[-1], W_down[-1])
    sbf_t = _sbf_t(sbf, W_sbf1[-1], W_sbf2[-1] * a32)
    xkt = _gather_mul_scatter(tmp2, sbf_t, idx_kj, idx_ji)
    return _post(xkt, xji, x, W_up, W_b1, b_b1, W_b2, b_b2,
                 W_lin, b_lin, W_a1, b_a1, W_a2, b_a2)


# speedup vs baseline: 1.4091x; 1.0006x over previous
import functools

import jax
import jax.numpy as jnp
from jax import lax
from jax.experimental import pallas as pl
from jax.experimental.pallas import tpu as pltpu
from jax.experimental.pallas import tpu_sc as plsc

E = 160000
T = 320000
H = 128
IE = 64

NCORE = 2
NSUB = 16
LANES = 16
NPASS = 3
RNG = 26752
ACC = RNG + 128
EP = NPASS * NCORE * RNG
CH = 80
TPT = T // NSUB
GSZ = 400
GCH = GSZ // CH
NGRP = TPT // GSZ
NCHT = TPT // CH
ZROWS = ACC // NSUB
OROWS = RNG // NSUB

BA = 3200
BB = 3200


def _silu(v):
    return v * jax.lax.logistic(v)


def _bdot(a, b):
    return jnp.dot(a.astype(jnp.bfloat16), b.astype(jnp.bfloat16),
                   preferred_element_type=jnp.float32)


def _pre_body(x_ref, rbf_ref, wji_ref, bji_ref, wkj_ref, bkj_ref,
              wr1_ref, wr2_ref, wd_ref, xji_ref, tmp2_ref):
    xb = x_ref[...]
    xji_ref[...] = _silu(_bdot(xb, wji_ref[...]) + bji_ref[...])
    t2 = _silu(_bdot(xb, wkj_ref[...]) + bkj_ref[...])
    r = _bdot(_bdot(rbf_ref[...], wr1_ref[...]), wr2_ref[...])
    tmp2_ref[...] = _silu(_bdot(t2 * r, wd_ref[...]))


def _sbf_body(sbf_ref, w1_ref, w2_ref, out_ref):
    out_ref[...] = _bdot(_bdot(sbf_ref[...], w1_ref[...]), w2_ref[...])


def _post_body(xkt_ref, xji_ref, x_ref, wup_ref, wb1_ref, bb1_ref, wb2_ref,
               bb2_ref, wlin_ref, blin_ref, wa1_ref, ba1_ref, wa2_ref,
               ba2_ref, out_ref):
    h0 = xji_ref[...] + _silu(_bdot(xkt_ref[...], wup_ref[...]))
    h1 = h0 + _silu(_bdot(_silu(_bdot(h0, wb1_ref[...]) + bb1_ref[...]),
                        wb2_ref[...]) + bb2_ref[...])
    h2 = _silu(_bdot(h1, wlin_ref[...]) + blin_ref[...]) + x_ref[...]
    out_ref[...] = h2 + _silu(
        _bdot(_silu(_bdot(h2, wa1_ref[...]) + ba1_ref[...]),
              wa2_ref[...]) + ba2_ref[...])


def _wspec(shape):
    return pl.BlockSpec(shape, lambda i: tuple(0 for _ in shape))


def _pre(x, rbf, w_ji, b_ji, w_kj, b_kj, w_r1, w_r2, w_d):
    nr = rbf.shape[1]
    return pl.pallas_call(
        _pre_body,
        grid=(E // BA,),
        in_specs=[
            pl.BlockSpec((BA, H), lambda i: (i, 0)),
            pl.BlockSpec((BA, nr), lambda i: (i, 0)),
            _wspec((H, H)), _wspec((1, H)), _wspec((H, H)), _wspec((1, H)),
            _wspec((nr, 8)), _wspec((8, H)), _wspec((H, IE)),
        ],
        out_specs=[
            pl.BlockSpec((BA, H), lambda i: (i, 0)),
            pl.BlockSpec((BA, IE), lambda i: (i, 0)),
        ],
        out_shape=[
            jax.ShapeDtypeStruct((E, H), jnp.float32),
            jax.ShapeDtypeStruct((E, IE), jnp.float32),
        ],
    )(x, rbf, w_ji, b_ji.reshape(1, H), w_kj, b_kj.reshape(1, H),
      w_r1, w_r2, w_d)


def _sbf_t(sbf, w1, w2):
    k = sbf.shape[1]
    return pl.pallas_call(
        _sbf_body,
        grid=(T // BB,),
        in_specs=[
            pl.BlockSpec((BB, k), lambda i: (i, 0)),
            _wspec((k, 8)), _wspec((8, IE)),
        ],
        out_specs=pl.BlockSpec((BB, IE), lambda i: (i, 0)),
        out_shape=jax.ShapeDtypeStruct((T, IE), jnp.float32),
    )(sbf, w1, w2)


def _post(xkt, xji, x, w_up, w_b1, b_b1, w_b2, b_b2, w_lin, b_lin,
          w_a1, b_a1, w_a2, b_a2):
    return pl.pallas_call(
        _post_body,
        grid=(E // BA,),
        in_specs=[
            pl.BlockSpec((BA, IE), lambda i: (i, 0)),
            pl.BlockSpec((BA, H), lambda i: (i, 0)),
            pl.BlockSpec((BA, H), lambda i: (i, 0)),
            _wspec((IE, H)),
            _wspec((H, H)), _wspec((1, H)), _wspec((H, H)), _wspec((1, H)),
            _wspec((H, H)), _wspec((1, H)),
            _wspec((H, H)), _wspec((1, H)), _wspec((H, H)), _wspec((1, H)),
        ],
        out_specs=pl.BlockSpec((BA, H), lambda i: (i, 0)),
        out_shape=jax.ShapeDtypeStruct((E, H), jnp.float32),
    )(xkt, xji, x, w_up,
      w_b1, b_b1.reshape(1, H), w_b2, b_b2.reshape(1, H),
      w_lin, b_lin.reshape(1, H),
      w_a1, b_a1.reshape(1, H), w_a2, b_a2.reshape(1, H))


def _sc_body(tmp2_hbm, sbft_hbm, kj_hbm, ji_hbm, zero_hbm, out_hbm,
             acc, semg0, semg1, sems0, sems1, semi0, semi1, semsc0, semsc1,
             kjg, jig, jilb, rows, sbfb):
    sid = lax.axis_index("s")
    cid = lax.axis_index("c")
    tbase = sid * TPT

    semg = (semg0, semg1)
    sems = (sems0, sems1)
    semi = (semi0, semi1)
    semsc = (semsc0, semsc1)

    def _idx_pair(g, gb):
        k = pltpu.make_async_copy(
            kj_hbm.at[pl.ds(tbase + g * GSZ, GSZ)], kjg.at[gb], semi[gb])
        j = pltpu.make_async_copy(
            ji_hbm.at[pl.ds(tbase + g * GSZ, GSZ)], jig.at[gb], semi[gb])
        return k, j

    def _gather(g, gb, ci, b):
        return pltpu.make_async_copy(
            tmp2_hbm.at[kjg.at[gb, pl.ds(ci * CH, CH)]], rows.at[b], semg[b])

    def _sbf(g, ci, b):
        return pltpu.make_async_copy(
            sbft_hbm.at[pl.ds(tbase + g * GSZ + ci * CH, CH)],
            sbfb.at[b], sems[b])

    def _scatter(b):
        return pltpu.make_async_copy(sbfb.at[b], acc.at[jilb.at[b]],
                                     semsc[b])

    def _nxt(g, gb, ci, d):
        ci2, g2, gb2 = ci + d, g, gb
        while ci2 >= GCH:
            ci2, g2, gb2 = ci2 - GCH, g2 + 1, 1 - gb2
        return g2, gb2, ci2

    def _chunk(g, gb, ci, b, base):
        t = g * GCH + ci
        gth = _gather(g, gb, ci, b)
        gth.wait()
        for k in range(CH // LANES):
            ji = jig[gb, pl.ds(ci * CH + k * LANES, LANES)]
            loc = ji - base
            inb = (loc >= 0) & (loc < RNG)
            jilb[b, pl.ds(k * LANES, LANES)] = jnp.where(inb, loc, RNG)
        _sbf(g, ci, b).wait()

        @plsc.parallel_loop(0, CH, 2, unroll=2)
        def _mul(jr):
            for jo in range(2):
                for k2 in range(IE // LANES):
                    sl = pl.ds(k2 * LANES, LANES)
                    sbfb[b, jr + jo, sl] = (
                        rows[b, jr + jo, sl] * sbfb[b, jr + jo, sl])

        _scatter(b).start(add=True)
        g2, gb2, ci2 = _nxt(g, gb, ci, 2)

        @pl.when(t + 2 < NCHT)
        def _():
            _gather(g2, gb2, ci2, b).start()

        @pl.when(t >= 1)
        def _():
            _scatter(1 - b).wait()
        g1, _gb1, ci1 = _nxt(g, gb, ci, 1)

        @pl.when(t + 1 < NCHT)
        def _():
            _sbf(g1, ci1, 1 - b).start()

    def _pass(p, carry):
        base = (p * NCORE + cid) * RNG

        pltpu.sync_copy(zero_hbm, acc.at[pl.ds(sid * ZROWS, ZROWS)])
        for d in _idx_pair(0, 0):
            d.start()
        for d in _idx_pair(0, 0):
            d.wait()
        for d in _idx_pair(1, 1):
            d.start()
        _gather(0, 0, 0, 0).start()
        _gather(0, 0, 1, 1).start()
        _sbf(0, 0, 0).start()
        plsc.subcore_barrier()

        def _group_pair(g2, cg):
            for gb in (0, 1):
                g = 2 * g2 + gb
                @pl.when(g + 1 < NGRP)
                def _():
                    for d in _idx_pair(g + 1, 1 - gb):
                        d.wait()

                for ci in range(GCH):
                    _chunk(g, gb, ci, (GCH * gb + ci) % 2, base)

                @pl.when(g + 2 < NGRP)
                def _():
                    for d in _idx_pair(g + 2, gb):
                        d.start()
            return cg

        lax.fori_loop(0, NGRP // 2, _group_pair, 0)
        _scatter(1).wait()
        plsc.subcore_barrier()
        pltpu.sync_copy(
            acc.at[pl.ds(sid * OROWS, OROWS)],
            out_hbm.at[pl.ds(base + sid * OROWS, OROWS)])
        return carry

    lax.fori_loop(0, NPASS, _pass, 0)


@functools.partial(
    pl.kernel,
    mesh=plsc.VectorSubcoreMesh(core_axis_name="c", subcore_axis_name="s"),
    compiler_params=pltpu.CompilerParams(use_tc_tiling_on_sc=False),
    out_type=jax.ShapeDtypeStruct((EP, IE), jnp.float32),
    scratch_types=[
        pltpu.VMEM_SHARED((ACC, IE), jnp.float32),
        pltpu.SemaphoreType.DMA,
        pltpu.SemaphoreType.DMA,
        pltpu.SemaphoreType.DMA,
        pltpu.SemaphoreType.DMA,
        pltpu.SemaphoreType.DMA,
        pltpu.SemaphoreType.DMA,
        pltpu.SemaphoreType.DMA,
        pltpu.SemaphoreType.DMA,
    ],
)
def _sc_gather_mul_scatter(tmp2_hbm, sbft_hbm, kj_hbm, ji_hbm, zero_hbm,
                           out_hbm, *rest):
    pl.run_scoped(
        lambda kjg, jig, jilb, rows, sbfb: _sc_body(
            tmp2_hbm, sbft_hbm, kj_hbm, ji_hbm, zero_hbm, out_hbm,
            *rest, kjg, jig, jilb, rows, sbfb),
        pltpu.VMEM((2, GSZ), jnp.int32),
        pltpu.VMEM((2, GSZ), jnp.int32),
        pltpu.VMEM((2, CH), jnp.int32),
        pltpu.VMEM((2, CH, IE), jnp.float32),
        pltpu.VMEM((2, CH, IE), jnp.float32),
    )


def _gather_mul_scatter(tmp2, sbf_t, idx_kj, idx_ji):
    zero = jnp.zeros((ZROWS, IE), jnp.float32)
    out = _sc_gather_mul_scatter(tmp2, sbf_t, idx_kj, idx_ji, zero)
    return out[:E]


def kernel(x, rbf, sbf, idx_kj, idx_ji, bt, lambda_d, alpha,
           W_rbf1, W_rbf2, W_sbf1, W_sbf2, W_kj, b_kj, W_ji, b_ji,
           W_down, W_up, W_b1, b_b1, W_b2, b_b2, W_lin, b_lin,
           W_a1, b_a1, W_a2, b_a2):
    a32 = jnp.asarray(alpha, jnp.float32)
    xji, tmp2 = _pre(x, rbf, W_ji, b_ji, W_kj[-1], b_kj[-1],
                     W_rbf1[-1], W_rbf2[-1], W_down[-1])
    sbf_t = _sbf_t(sbf, W_sbf1[-1], W_sbf2[-1] * a32)
    xkt = _gather_mul_scatter(tmp2, sbf_t, idx_kj, idx_ji)
    return _post(xkt, xji, x, W_up, W_b1, b_b1, W_b2, b_b2,
                 W_lin, b_lin, W_a1, b_a1, W_a2, b_a2)
